# baseline (speedup 1.0000x reference)
import functools

import jax
import jax.numpy as jnp
from jax import lax
from jax.experimental import pallas as pl
from jax.experimental.pallas import tpu as pltpu
from jax.experimental.pallas import tpu_sc as plsc

CHUNK = 128


@functools.partial(jax.jit, static_argnames=("n_chunks", "d"))
def _sc_gather(weight, idx3, *, n_chunks, d):
    nw = idx3.shape[0]
    b = nw * n_chunks * CHUNK
    mesh = plsc.VectorSubcoreMesh(core_axis_name="c", subcore_axis_name="s")

    @functools.partial(
        pl.kernel,
        mesh=mesh,
        compiler_params=pltpu.CompilerParams(use_tc_tiling_on_sc=False),
        out_type=jax.ShapeDtypeStruct((b, d), jnp.float32),
        scratch_types=[
            pltpu.VMEM((n_chunks, CHUNK), jnp.int32),
            pltpu.VMEM((2, CHUNK, d), jnp.float32),
            pltpu.SemaphoreType.DMA,
            pltpu.SemaphoreType.DMA,
        ],
    )
    def k(table_hbm, idx_hbm, out_hbm, idx_v, rows_v, gsem, ssem):
        wid = lax.axis_index("s") * 2 + lax.axis_index("c")
        base = wid * (n_chunks * CHUNK)
        pltpu.sync_copy(idx_hbm.at[wid], idx_v)

        pltpu.async_copy(table_hbm.at[idx_v.at[0]], rows_v.at[0], gsem).wait()

        def body(j, _):
            buf = lax.rem(j, 2)

            @pl.when(j + 1 < n_chunks)
            def _():
                pltpu.async_copy(
                    table_hbm.at[idx_v.at[j + 1]], rows_v.at[1 - buf], gsem
                )

            pltpu.async_copy(
                rows_v.at[buf], out_hbm.at[pl.ds(base + j * CHUNK, CHUNK)], ssem
            ).wait()

            @pl.when(j + 1 < n_chunks)
            def _():
                pltpu.make_async_copy(
                    table_hbm.at[idx_v.at[j + 1]], rows_v.at[1 - buf], gsem
                ).wait()

            return 0

        lax.fori_loop(0, n_chunks, body, 0)

    return k(weight, idx3)


def kernel(token_ids, weight):
    bt, s = token_ids.shape
    d = weight.shape[1]
    flat = token_ids.reshape(-1).astype(jnp.int32)
    b = flat.shape[0]
    nw = 32
    per_w = b // nw
    n_chunks = per_w // CHUNK
    idx3 = flat.reshape(nw, n_chunks, CHUNK)
    out = _sc_gather(weight, idx3, n_chunks=n_chunks, d=d)
    return out.reshape(bt, s, d)


# speedup vs baseline: 1.0738x; 1.0738x over previous
import functools

import jax
import jax.numpy as jnp
from jax import lax
from jax.experimental import pallas as pl
from jax.experimental.pallas import tpu as pltpu
from jax.experimental.pallas import tpu_sc as plsc

CHUNK = 128
NBUF = 8
LA = 4


@functools.partial(jax.jit, static_argnames=("n_chunks", "d"))
def _sc_gather(weight, idx3, *, n_chunks, d):
    nw = idx3.shape[0]
    b = nw * n_chunks * CHUNK
    mesh = plsc.VectorSubcoreMesh(core_axis_name="c", subcore_axis_name="s")
    n_outer = n_chunks // NBUF

    @functools.partial(
        pl.kernel,
        mesh=mesh,
        compiler_params=pltpu.CompilerParams(use_tc_tiling_on_sc=False),
        out_type=jax.ShapeDtypeStruct((b, d), jnp.float32),
        scratch_types=(
            [pltpu.VMEM((n_chunks, CHUNK), jnp.int32),
             pltpu.VMEM((NBUF, CHUNK, d), jnp.float32)]
            + [pltpu.SemaphoreType.DMA] * (2 * NBUF)
        ),
    )
    def k(table_hbm, idx_hbm, out_hbm, idx_v, rows_v, *sems):
        gsem = sems[:NBUF]
        ssem = sems[NBUF:]
        wid = lax.axis_index("s") * 2 + lax.axis_index("c")
        base = wid * (n_chunks * CHUNK)
        pltpu.sync_copy(idx_hbm.at[wid], idx_v)

        for jj in range(LA):
            pltpu.async_copy(table_hbm.at[idx_v.at[jj]], rows_v.at[jj], gsem[jj])

        def outer(g, _):
            j0 = g * NBUF
            for bb in range(NBUF):
                j = j0 + bb
                jl = j + LA
                bl = (bb + LA) % NBUF

                @pl.when(jl < n_chunks)
                def _():
                    @pl.when(jl >= NBUF)
                    def _():
                        pltpu.make_async_copy(
                            rows_v.at[bl],
                            out_hbm.at[pl.ds(base, CHUNK)],
                            ssem[bl],
                        ).wait()

                    pltpu.async_copy(
                        table_hbm.at[idx_v.at[jl]], rows_v.at[bl], gsem[bl]
                    )

                pltpu.make_async_copy(
                    table_hbm.at[idx_v.at[j]], rows_v.at[bb], gsem[bb]
                ).wait()
                pltpu.async_copy(
                    rows_v.at[bb], out_hbm.at[pl.ds(base + j * CHUNK, CHUNK)], ssem[bb]
                )
            return 0

        lax.fori_loop(0, n_outer, outer, 0)

        for bb in range(NBUF):
            pltpu.make_async_copy(
                rows_v.at[bb], out_hbm.at[pl.ds(base, CHUNK)], ssem[bb]
            ).wait()

    return k(weight, idx3)


def kernel(token_ids, weight):
    bt, s = token_ids.shape
    d = weight.shape[1]
    flat = token_ids.reshape(-1).astype(jnp.int32)
    b = flat.shape[0]
    nw = 32
    per_w = b // nw
    n_chunks = per_w // CHUNK
    idx3 = flat.reshape(nw, n_chunks, CHUNK)
    out = _sc_gather(weight, idx3, n_chunks=n_chunks, d=d)
    return out.reshape(bt, s, d)
